# Initial kernel scaffold; baseline (speedup 1.0000x reference)
import jax
import jax.numpy as jnp
from jax.experimental import pallas as pl


def kernel(sentences, lengths, bert_sent, bert_sent_type, bert_sent_mask, embed_table):
    raise NotImplementedError("write your pallas kernel here")


# speedup vs baseline: 4.0865x; 4.0865x over previous
import functools

import jax
import jax.numpy as jnp
from jax import lax
from jax.experimental import pallas as pl
from jax.experimental.pallas import tpu as pltpu
from jax.experimental.pallas import tpu_sc as plsc

_NC = 2
_NS = 16
_NW = _NC * _NS

_CHUNK = 128


@functools.lru_cache(maxsize=None)
def _build_gather(N: int, V: int, D: int):
    n_per_w = N // _NW
    n_chunks = n_per_w // _CHUNK
    mesh = plsc.VectorSubcoreMesh(core_axis_name="c", subcore_axis_name="s")

    @functools.partial(
        pl.kernel,
        out_type=jax.ShapeDtypeStruct((N, D), jnp.float32),
        mesh=mesh,
        scratch_types=[
            pltpu.VMEM((n_chunks, _CHUNK), jnp.int32),
            pltpu.VMEM((_CHUNK, D), jnp.float32),
            pltpu.SemaphoreType.DMA,
        ],
        compiler_params=pltpu.CompilerParams(use_tc_tiling_on_sc=False),
    )
    def gather_kernel(table_hbm, idx_hbm, out_hbm, idx_v, rows_v, sem):
        wid = lax.axis_index("s") * _NC + lax.axis_index("c")
        base = wid * n_per_w
        pltpu.sync_copy(idx_hbm.at[wid], idx_v)

        def chunk(j, carry):
            pltpu.async_copy(table_hbm.at[idx_v.at[j]], rows_v, sem).wait()
            pltpu.sync_copy(rows_v, out_hbm.at[pl.ds(base + j * _CHUNK, _CHUNK)])
            return carry

        lax.fori_loop(0, n_chunks, chunk, 0)

    return gather_kernel


def kernel(sentences, lengths, bert_sent, bert_sent_type, bert_sent_mask, embed_table):
    B, L = sentences.shape
    V, D = embed_table.shape
    N = B * L
    idx = sentences.astype(jnp.int32).reshape(_NW, N // (_NW * _CHUNK), _CHUNK)
    out = _build_gather(N, V, D)(embed_table, idx)
    return out.reshape(B, L, D)


# speedup vs baseline: 4.6753x; 1.1441x over previous
import functools

import jax
import jax.numpy as jnp
from jax import lax
from jax.experimental import pallas as pl
from jax.experimental.pallas import tpu as pltpu
from jax.experimental.pallas import tpu_sc as plsc

_NC = 2
_NS = 16
_NW = _NC * _NS

_CHUNK = 128
_NBUF = 5


@functools.lru_cache(maxsize=None)
def _build_gather(N: int, V: int, D: int):
    n_per_w = N // _NW
    n_chunks = n_per_w // _CHUNK
    n_groups = n_chunks // _NBUF
    mesh = plsc.VectorSubcoreMesh(core_axis_name="c", subcore_axis_name="s")

    @functools.partial(
        pl.kernel,
        out_type=jax.ShapeDtypeStruct((N, D), jnp.float32),
        mesh=mesh,
        scratch_types=[
            pltpu.VMEM((n_chunks, _CHUNK), jnp.int32),
            [pltpu.VMEM((_CHUNK, D), jnp.float32) for _ in range(_NBUF)],
            [pltpu.SemaphoreType.DMA for _ in range(_NBUF)],
            [pltpu.SemaphoreType.DMA for _ in range(_NBUF)],
        ],
        compiler_params=pltpu.CompilerParams(use_tc_tiling_on_sc=False),
    )
    def gather_kernel(table_hbm, idx_hbm, out_hbm, idx_v, rows, gsem, wsem):
        wid = lax.axis_index("s") * _NC + lax.axis_index("c")
        base = wid * n_per_w
        pltpu.sync_copy(idx_hbm.at[wid], idx_v)

        for b in range(_NBUF):
            pltpu.async_copy(table_hbm.at[idx_v.at[b]], rows[b], gsem[b])

        def group(g, carry):
            for b in range(_NBUF):
                j = g * _NBUF + b
                pltpu.make_async_copy(
                    table_hbm.at[idx_v.at[j]], rows[b], gsem[b]
                ).wait()
                dst = out_hbm.at[pl.ds(base + j * _CHUNK, _CHUNK)]
                pltpu.async_copy(rows[b], dst, wsem[b])

                @pl.when(g < n_groups - 1)
                def _():
                    pltpu.make_async_copy(rows[b], dst, wsem[b]).wait()
                    pltpu.async_copy(
                        table_hbm.at[idx_v.at[j + _NBUF]], rows[b], gsem[b]
                    )

            return carry

        lax.fori_loop(0, n_groups, group, 0)

        for b in range(_NBUF):
            j = (n_groups - 1) * _NBUF + b
            dst = out_hbm.at[pl.ds(base + j * _CHUNK, _CHUNK)]
            pltpu.make_async_copy(rows[b], dst, wsem[b]).wait()

    return gather_kernel


def kernel(sentences, lengths, bert_sent, bert_sent_type, bert_sent_mask, embed_table):
    B, L = sentences.shape
    V, D = embed_table.shape
    N = B * L
    idx = sentences.astype(jnp.int32).reshape(_NW, N // (_NW * _CHUNK), _CHUNK)
    out = _build_gather(N, V, D)(embed_table, idx)
    return out.reshape(B, L, D)
